# baseline (speedup 1.0000x reference)
import functools
import math

import jax
import jax.numpy as jnp
from jax import lax
from jax.experimental import pallas as pl
from jax.experimental.pallas import tpu as pltpu
from jax.experimental.pallas import tpu_sc as plsc

_NUM_BUCKETS = 64
_MAX_DISTANCE = 32768
_TABLE_ROWS = 256
_TABLE_COLS = 128

_L = 16
_NW = 32
_N = 64 * 32768
_PER_W = _N // _NW
_CH = 16384
_NCH = _PER_W // _CH


def _table_body(bias_ref, out_ref):
    r = lax.broadcasted_iota(jnp.int32, (_TABLE_ROWS, _TABLE_COLS), 0)
    c = lax.broadcasted_iota(jnp.int32, (_TABLE_ROWS, _TABLE_COLS), 1)
    p = r * _TABLE_COLS + c
    exact = _NUM_BUCKETS // 2
    rel = p.astype(jnp.float32) - exact
    log_b = exact + (_NUM_BUCKETS - exact - 1) * jnp.log(
        jnp.clip(rel, 1.0, None)) / math.log(max(_MAX_DISTANCE - exact, 2))
    bucket = jnp.where(p < exact, p, log_b.astype(jnp.int32))
    bucket = jnp.clip(bucket, 0, _NUM_BUCKETS - 1)
    acc = jnp.zeros((_TABLE_ROWS, _TABLE_COLS), jnp.float32)
    for b in range(_NUM_BUCKETS):
        acc = jnp.where(bucket == b, bias_ref[b], acc)
    out_ref[...] = acc


def _build_table(bias):
    table2d = pl.pallas_call(
        _table_body,
        out_shape=jax.ShapeDtypeStruct((_TABLE_ROWS, _TABLE_COLS), jnp.float32),
        in_specs=[pl.BlockSpec(memory_space=pltpu.SMEM)],
        out_specs=pl.BlockSpec(memory_space=pltpu.VMEM),
    )(bias)
    return table2d.reshape(_MAX_DISTANCE)


_ROWS = 64
_COLS = 32768
_ROWS_PER_W = _ROWS // _NW
_CH_PER_ROW = _COLS // _CH
_NCHUNK = _ROWS_PER_W * _CH_PER_ROW


_S = 3584
_V = _CH - _S


def _sc_gather(table, positions):
    mesh = plsc.VectorSubcoreMesh(core_axis_name="c", subcore_axis_name="s")

    @functools.partial(
        pl.kernel,
        mesh=mesh,
        out_type=jax.ShapeDtypeStruct((_ROWS, _COLS), jnp.float32),
        compiler_params=pltpu.CompilerParams(
            needs_layout_passes=False,
            disable_bounds_checks=True,
            disable_semaphore_checks=True,
        ),
        scratch_types=[
            pltpu.VMEM((_MAX_DISTANCE,), jnp.float32),
            pltpu.VMEM_SHARED((_MAX_DISTANCE,), jnp.float32),
            pltpu.VMEM((_V,), jnp.int32),
            pltpu.VMEM((_V,), jnp.int32),
            pltpu.VMEM((_S,), jnp.int32),
            pltpu.VMEM((_S,), jnp.int32),
            pltpu.VMEM((_V,), jnp.float32),
            pltpu.VMEM((_V,), jnp.float32),
            pltpu.VMEM((_S,), jnp.float32),
            pltpu.VMEM((_S,), jnp.float32),
            pltpu.SemaphoreType.DMA,
            pltpu.SemaphoreType.DMA,
            pltpu.SemaphoreType.DMA,
            pltpu.SemaphoreType.DMA,
            pltpu.SemaphoreType.DMA,
            pltpu.SemaphoreType.DMA,
            pltpu.SemaphoreType.DMA,
            pltpu.SemaphoreType.DMA,
            pltpu.SemaphoreType.DMA,
            pltpu.SemaphoreType.DMA,
            pltpu.SemaphoreType.DMA,
            pltpu.SemaphoreType.DMA,
        ],
    )
    def k(table_hbm, pos_hbm, out_hbm, table_v, table_sh,
          idx0, idx1, sidx0, sidx1, val0, val1, sval0, sval1,
          tsem, shsem, lsem0, lsem1, slsem0, slsem1,
          ssem0, ssem1, sssem0, sssem1, gsem0, gsem1):
        sid = lax.axis_index("s")
        wid = sid * 2 + lax.axis_index("c")
        idx = [idx0, idx1]
        sidx = [sidx0, sidx1]
        val = [val0, val1]
        sval = [sval0, sval1]
        lsem = [lsem0, lsem1]
        slsem = [slsem0, slsem1]
        ssem = [ssem0, ssem1]
        sssem = [sssem0, sssem1]
        gsem = [gsem0, gsem1]

        def pos_slice(c):
            row = wid * _ROWS_PER_W + (c // _CH_PER_ROW)
            col = (c % _CH_PER_ROW) * _CH
            return row, col

        def issue_loads(c, b):
            r, col = pos_slice(c)
            pltpu.async_copy(pos_hbm.at[r, pl.ds(col, _S)], sidx[b], slsem[b])
            pltpu.async_copy(pos_hbm.at[r, pl.ds(col + _S, _V)], idx[b], lsem[b])

        ht = pltpu.async_copy(table_hbm, table_v, tsem)
        for c in range(2):
            issue_loads(c, c % 2)

        @pl.when(sid == 0)
        def _():
            pltpu.async_copy(table_hbm, table_sh, shsem).wait()
        plsc.subcore_barrier()
        ht.wait()

        pltpu.make_async_copy(
            pos_hbm.at[0, pl.ds(0, _S)], sidx[0], slsem[0]).wait()
        pltpu.async_copy(table_sh.at[sidx[0]], sval[0], gsem[0])

        def super_body(s_i, carry):
            for b in range(2):
                c = s_i * 2 + b
                bn = (b + 1) % 2

                @pl.when(c + 1 < _NCHUNK)
                def _():
                    pltpu.make_async_copy(
                        pos_hbm.at[0, pl.ds(0, _S)], sidx[bn], slsem[bn]).wait()

                    @pl.when(c >= 1)
                    def _():
                        pltpu.make_async_copy(
                            sval[bn], out_hbm.at[0, pl.ds(0, _S)], sssem[bn]).wait()

                    pltpu.async_copy(table_sh.at[sidx[bn]], sval[bn], gsem[bn])

                pltpu.make_async_copy(
                    pos_hbm.at[0, pl.ds(0, _V)], idx[b], lsem[b]).wait()

                @pl.when(c >= 2)
                def _():
                    pltpu.make_async_copy(
                        val[b], out_hbm.at[0, pl.ds(0, _V)], ssem[b]).wait()

                @plsc.parallel_loop(0, _V, step=_L, unroll=8)
                def gather_body(i, _idx=idx[b], _val=val[b]):
                    _val[pl.ds(i, _L)] = plsc.load_gather(table_v, [_idx[pl.ds(i, _L)]])

                pltpu.make_async_copy(
                    table_sh.at[sidx[b]], sval[b], gsem[b]).wait()
                r, col = pos_slice(c)
                pltpu.async_copy(sval[b], out_hbm.at[r, pl.ds(col, _S)], sssem[b])
                pltpu.async_copy(val[b], out_hbm.at[r, pl.ds(col + _S, _V)], ssem[b])

                @pl.when(c + 2 < _NCHUNK)
                def _():
                    issue_loads(c + 2, b)
            return carry

        lax.fori_loop(0, _NCHUNK // 2, super_body, 0)

        for b in range(2):
            pltpu.make_async_copy(
                sval[b], out_hbm.at[0, pl.ds(0, _S)], sssem[b]).wait()
            pltpu.make_async_copy(
                val[b], out_hbm.at[0, pl.ds(0, _V)], ssem[b]).wait()

    return k(table, positions)


def kernel(positions, bias):
    table = _build_table(bias)
    return _sc_gather(table, positions)


# speedup vs baseline: 1.0802x; 1.0802x over previous
import functools
import math

import jax
import jax.numpy as jnp
from jax import lax
from jax.experimental import pallas as pl
from jax.experimental.pallas import tpu as pltpu
from jax.experimental.pallas import tpu_sc as plsc

_NUM_BUCKETS = 64
_MAX_DISTANCE = 32768
_TABLE_ROWS = 256
_TABLE_COLS = 128

_L = 16
_NW = 32
_N = 64 * 32768
_PER_W = _N // _NW
_CH = 16384
_NCH = _PER_W // _CH


def _table_body(bias_ref, out_ref):
    r = lax.broadcasted_iota(jnp.int32, (_TABLE_ROWS, _TABLE_COLS), 0)
    c = lax.broadcasted_iota(jnp.int32, (_TABLE_ROWS, _TABLE_COLS), 1)
    p = r * _TABLE_COLS + c
    exact = _NUM_BUCKETS // 2
    rel = p.astype(jnp.float32) - exact
    log_b = exact + (_NUM_BUCKETS - exact - 1) * jnp.log(
        jnp.clip(rel, 1.0, None)) / math.log(max(_MAX_DISTANCE - exact, 2))
    bucket = jnp.where(p < exact, p, log_b.astype(jnp.int32))
    bucket = jnp.clip(bucket, 0, _NUM_BUCKETS - 1)
    acc = jnp.zeros((_TABLE_ROWS, _TABLE_COLS), jnp.float32)
    for b in range(_NUM_BUCKETS):
        acc = jnp.where(bucket == b, bias_ref[b], acc)
    out_ref[...] = acc


def _build_table(bias):
    table2d = pl.pallas_call(
        _table_body,
        out_shape=jax.ShapeDtypeStruct((_TABLE_ROWS, _TABLE_COLS), jnp.float32),
        in_specs=[pl.BlockSpec(memory_space=pltpu.SMEM)],
        out_specs=pl.BlockSpec(memory_space=pltpu.VMEM),
    )(bias)
    return table2d.reshape(_MAX_DISTANCE)


_ROWS = 64
_COLS = 32768
_ROWS_PER_W = _ROWS // _NW
_CH_PER_ROW = _COLS // _CH
_NCHUNK = _ROWS_PER_W * _CH_PER_ROW


_S = 3584
_V = _CH - _S


def _sc_gather(table, positions):
    mesh = plsc.VectorSubcoreMesh(core_axis_name="c", subcore_axis_name="s")

    @functools.partial(
        pl.kernel,
        mesh=mesh,
        out_type=jax.ShapeDtypeStruct((_ROWS, _COLS), jnp.float32),
        compiler_params=pltpu.CompilerParams(
            needs_layout_passes=False,
            disable_bounds_checks=True,
            disable_semaphore_checks=True,
        ),
        scratch_types=[
            pltpu.VMEM((_MAX_DISTANCE,), jnp.float32),
            pltpu.VMEM_SHARED((_MAX_DISTANCE,), jnp.float32),
            pltpu.VMEM((_V,), jnp.int32),
            pltpu.VMEM((_V,), jnp.int32),
            pltpu.VMEM((_S,), jnp.int32),
            pltpu.VMEM((_S,), jnp.int32),
            pltpu.VMEM((_V,), jnp.float32),
            pltpu.VMEM((_V,), jnp.float32),
            pltpu.VMEM((_S,), jnp.float32),
            pltpu.VMEM((_S,), jnp.float32),
            pltpu.SemaphoreType.DMA,
            pltpu.SemaphoreType.DMA,
            pltpu.SemaphoreType.DMA,
            pltpu.SemaphoreType.DMA,
            pltpu.SemaphoreType.DMA,
            pltpu.SemaphoreType.DMA,
            pltpu.SemaphoreType.DMA,
            pltpu.SemaphoreType.DMA,
            pltpu.SemaphoreType.DMA,
            pltpu.SemaphoreType.DMA,
            pltpu.SemaphoreType.DMA,
            pltpu.SemaphoreType.DMA,
        ],
    )
    def k(table_hbm, pos_hbm, out_hbm, table_v, table_sh,
          idx0, idx1, sidx0, sidx1, val0, val1, sval0, sval1,
          tsem, shsem, lsem0, lsem1, slsem0, slsem1,
          ssem0, ssem1, sssem0, sssem1, gsem0, gsem1):
        sid = lax.axis_index("s")
        wid = sid * 2 + lax.axis_index("c")
        idx = [idx0, idx1]
        sidx = [sidx0, sidx1]
        val = [val0, val1]
        sval = [sval0, sval1]
        lsem = [lsem0, lsem1]
        slsem = [slsem0, slsem1]
        ssem = [ssem0, ssem1]
        sssem = [sssem0, sssem1]
        gsem = [gsem0, gsem1]

        def pos_slice(c):
            row = wid * _ROWS_PER_W + (c // _CH_PER_ROW)
            col = (c % _CH_PER_ROW) * _CH
            return row, col

        def issue_loads(c, b):
            r, col = pos_slice(c)
            pltpu.async_copy(pos_hbm.at[r, pl.ds(col, _S)], sidx[b], slsem[b])
            pltpu.async_copy(pos_hbm.at[r, pl.ds(col + _S, _V)], idx[b], lsem[b])

        ht = pltpu.async_copy(table_hbm, table_v, tsem)
        for c in range(2):
            issue_loads(c, c % 2)

        @pl.when(sid == 0)
        def _():
            pltpu.async_copy(table_hbm, table_sh, shsem).wait()
        plsc.subcore_barrier()
        ht.wait()

        def super_body(s_i, carry):
            for b in range(2):
                c = s_i * 2 + b
                pltpu.make_async_copy(
                    pos_hbm.at[0, pl.ds(0, _S)], sidx[b], slsem[b]).wait()

                @pl.when(c >= 2)
                def _():
                    pltpu.make_async_copy(
                        sval[b], out_hbm.at[0, pl.ds(0, _S)], sssem[b]).wait()

                hg = pltpu.async_copy(table_sh.at[sidx[b]], sval[b], gsem[b])
                pltpu.make_async_copy(
                    pos_hbm.at[0, pl.ds(0, _V)], idx[b], lsem[b]).wait()

                @pl.when(c >= 2)
                def _():
                    pltpu.make_async_copy(
                        val[b], out_hbm.at[0, pl.ds(0, _V)], ssem[b]).wait()

                @plsc.parallel_loop(0, _V, step=_L, unroll=8)
                def gather_body(i, _idx=idx[b], _val=val[b]):
                    _val[pl.ds(i, _L)] = plsc.load_gather(table_v, [_idx[pl.ds(i, _L)]])

                hg.wait()
                r, col = pos_slice(c)
                pltpu.async_copy(sval[b], out_hbm.at[r, pl.ds(col, _S)], sssem[b])
                pltpu.async_copy(val[b], out_hbm.at[r, pl.ds(col + _S, _V)], ssem[b])

                @pl.when(c + 2 < _NCHUNK)
                def _():
                    issue_loads(c + 2, b)
            return carry

        lax.fori_loop(0, _NCHUNK // 2, super_body, 0)

        for b in range(2):
            pltpu.make_async_copy(
                sval[b], out_hbm.at[0, pl.ds(0, _S)], sssem[b]).wait()
            pltpu.make_async_copy(
                val[b], out_hbm.at[0, pl.ds(0, _V)], ssem[b]).wait()

    return k(table, positions)


def kernel(positions, bias):
    table = _build_table(bias)
    return _sc_gather(table, positions)
